# baseline (device time: 164032 ns/iter reference)
import os

import jax
import jax.numpy as jnp
from jax import lax
from jax.experimental import pallas as pl
from jax.experimental.pallas import tpu as pltpu

_SKIP_COMM = os.environ.get("KERNEL_SKIP_COMM") == "1"

N_DEV = 8
AXIS = "i"
M = 64
LAYERS = 3
T = 8
HT = 512
SLOTS = 3

_COLS = ((0, 768), (768, 640), (1408, 640))

_RS_BITS = ((1, 3, 4), (3, 4, 1), (4, 1, 3))
_AG_BITS = ((4, 3, 1), (1, 4, 3), (3, 1, 4))


def _rs_send_sets(bits):
    out = []
    for k in range(3):
        span = [0]
        for b in bits[k + 1:]:
            span = span + [s ^ b for s in span]
        out.append(tuple(bits[k] ^ s for s in span))
    return tuple(out)


def _ag_send_sets(bits):
    held = [0]
    out = []
    for b in bits:
        out.append(tuple(held))
        held = held + [h ^ b for h in held]
    return tuple(out)


_RS_SENDS = tuple(_rs_send_sets(b) for b in _RS_BITS)
_AG_SENDS = tuple(_ag_send_sets(b) for b in _AG_BITS)
_RS_OFF = (0, 4, 6)
_AG_SEM_BASE = (0, 1, 3)


def _barrier(d):
    sem = pltpu.get_barrier_semaphore()
    for b in (1, 3, 4):
        pl.semaphore_signal(sem, inc=1, device_id=(jnp.bitwise_xor(d, b),),
                            device_id_type=pl.DeviceIdType.MESH)
    pl.semaphore_wait(sem, 3)


def _hd_allgather(out_ref, d, send_sems, recv_sems):
    for step in range(3):
        rdmas = []
        for sched, (c0, cw) in enumerate(_COLS):
            bit = _AG_BITS[sched][step]
            q = jnp.bitwise_xor(d, bit)
            for slot, j in enumerate(_AG_SENDS[sched][step]):
                c = jnp.bitwise_xor(d, j)
                sl = _AG_SEM_BASE[step] + slot
                rdma = pltpu.make_async_remote_copy(
                    src_ref=out_ref.at[pl.ds(c * M, M), pl.ds(c0, cw)],
                    dst_ref=out_ref.at[pl.ds(c * M, M), pl.ds(c0, cw)],
                    send_sem=send_sems.at[sched, sl],
                    recv_sem=recv_sems.at[sched, sl],
                    device_id=(q,),
                    device_id_type=pl.DeviceIdType.MESH,
                )
                rdma.start()
                rdmas.append(rdma)
        for r in rdmas:
            r.wait()


def _hd_reduce_scatter(acc_ref, snd_ref, rcv_ref, d, send_sems, recv_sems):
    for step in range(3):
        rdmas = []
        for sched, (c0, cw) in enumerate(_COLS):
            bit = _RS_BITS[sched][step]
            j_list = _RS_SENDS[sched][step]
            q = jnp.bitwise_xor(d, bit)
            off = _RS_OFF[step] * M
            for slot, j in enumerate(j_list):
                c = jnp.bitwise_xor(d, j)
                snd_ref[sched, pl.ds(off + slot * M, M), 0:cw] = acc_ref[
                    pl.ds(c * M, M), pl.ds(c0, cw)
                ].astype(jnp.bfloat16)
            nrows = len(j_list) * M
            rdma = pltpu.make_async_remote_copy(
                src_ref=snd_ref.at[sched, pl.ds(off, nrows), 0:cw],
                dst_ref=rcv_ref.at[sched, pl.ds(off, nrows), 0:cw],
                send_sem=send_sems.at[sched, step],
                recv_sem=recv_sems.at[sched, step],
                device_id=(q,),
                device_id_type=pl.DeviceIdType.MESH,
            )
            rdma.start()
            rdmas.append(rdma)
        for r in rdmas:
            r.wait()
        for sched, (c0, cw) in enumerate(_COLS):
            bit = _RS_BITS[sched][step]
            off = _RS_OFF[step] * M
            for slot, j in enumerate(_RS_SENDS[sched][step]):
                c = jnp.bitwise_xor(d, bit ^ j)
                acc_ref[pl.ds(c * M, M), pl.ds(c0, cw)] += rcv_ref[
                    sched, pl.ds(off + slot * M, M), 0:cw
                ].astype(jnp.float32)


def kernel(x, Win0, Wout0, Win1, Wout1, Win2, Wout2):
    m, n = x.shape
    B = N_DEV * m

    def body(x_ref, wi0, wo0, wi1, wo1, wi2, wo2, out_ref,
             xf_ref, acc_ref, wi_buf, wo_buf, snd_ref, rcv_ref,
             wi_sems, wo_sems, rs_ssem, rs_rsem, ag_ssem, ag_rsem):
        d = lax.axis_index(AXIS)
        if not _SKIP_COMM:
            _barrier(d)

        wins = [wi0, wi1, wi2]
        wouts = [wo0, wo1, wo2]
        copies = {}

        def issue(g):
            if g >= LAYERS * T:
                return
            L, t = divmod(g, T)
            s = g % SLOTS
            ci = pltpu.make_async_copy(
                wins[L].at[:, pl.ds(t * HT, HT)], wi_buf.at[s], wi_sems.at[s])
            co = pltpu.make_async_copy(
                wouts[L].at[pl.ds(t * HT, HT), :], wo_buf.at[s], wo_sems.at[s])
            ci.start()
            co.start()
            copies[g] = (ci, co)

        for g in range(SLOTS):
            issue(g)

        if _SKIP_COMM:
            for c in range(N_DEV):
                xf_ref[pl.ds(c * m, m), :] = x_ref[...].astype(jnp.bfloat16)
        else:
            xf_ref[pl.ds(d * m, m), :] = x_ref[...].astype(jnp.bfloat16)
            _hd_allgather(xf_ref, d, ag_ssem, ag_rsem)

        def load_cast(g):
            ci, co = copies.pop(g)
            ci.wait()
            co.wait()
            s = g % SLOTS
            return (wi_buf[s].astype(jnp.bfloat16),
                    wo_buf[s].astype(jnp.bfloat16))

        nxt = load_cast(0)
        for L in range(LAYERS):
            for t in range(T):
                g = L * T + t
                wbi, wbo = nxt
                h = jnp.dot(xf_ref[...], wbi,
                            preferred_element_type=jnp.float32)
                h = jnp.maximum(h, 0.0).astype(jnp.bfloat16)
                p = jnp.dot(h, wbo, preferred_element_type=jnp.float32)
                issue(g + SLOTS)
                if g + 1 < LAYERS * T:
                    nxt = load_cast(g + 1)
                if t == 0:
                    acc_ref[...] = p
                else:
                    acc_ref[...] += p

            if _SKIP_COMM:
                xf_ref[...] = acc_ref[...].astype(jnp.bfloat16)
            else:
                _hd_reduce_scatter(acc_ref, snd_ref, rcv_ref, d,
                                   rs_ssem, rs_rsem)
                xf_ref[pl.ds(d * M, M), :] = acc_ref[
                    pl.ds(d * M, M), :
                ].astype(jnp.bfloat16)
                _hd_allgather(xf_ref, d, ag_ssem, ag_rsem)

        out_ref[...] = xf_ref[...].astype(jnp.float32)

    any_spec = pl.BlockSpec(memory_space=pl.ANY)
    return pl.pallas_call(
        body,
        out_shape=jax.ShapeDtypeStruct((B, n), jnp.float32),
        in_specs=[pl.BlockSpec(memory_space=pltpu.VMEM)] + [any_spec] * 6,
        out_specs=pl.BlockSpec(memory_space=pltpu.VMEM),
        scratch_shapes=[
            pltpu.VMEM((B, n), jnp.bfloat16),
            pltpu.VMEM((B, n), jnp.float32),
            pltpu.VMEM((SLOTS, n, HT), jnp.float32),
            pltpu.VMEM((SLOTS, HT, n), jnp.float32),
            pltpu.VMEM((3, 7 * M, 768), jnp.bfloat16),
            pltpu.VMEM((3, 7 * M, 768), jnp.bfloat16),
            pltpu.SemaphoreType.DMA((SLOTS,)),
            pltpu.SemaphoreType.DMA((SLOTS,)),
            pltpu.SemaphoreType.DMA((3, 3)),
            pltpu.SemaphoreType.DMA((3, 3)),
            pltpu.SemaphoreType.DMA((3, 7)),
            pltpu.SemaphoreType.DMA((3, 7)),
        ],
        compiler_params=pltpu.CompilerParams(
            collective_id=None if _SKIP_COMM else 0,
            vmem_limit_bytes=60 * 1024 * 1024),
    )(x, Win0, Wout0, Win1, Wout1, Win2, Wout2)


# device time: 163564 ns/iter; 1.0029x vs baseline; 1.0029x over previous
import os

import jax
import jax.numpy as jnp
from jax import lax
from jax.experimental import pallas as pl
from jax.experimental.pallas import tpu as pltpu

_SKIP_COMM = os.environ.get("KERNEL_SKIP_COMM") == "1"

N_DEV = 8
AXIS = "i"
M = 64
LAYERS = 3
T = 8
HT = 512
SLOTS = 4

_COLS = ((0, 768), (768, 640), (1408, 640))

_RS_BITS = ((1, 3, 4), (3, 4, 1), (4, 1, 3))
_AG_BITS = ((4, 3, 1), (1, 4, 3), (3, 1, 4))


def _rs_send_sets(bits):
    out = []
    for k in range(3):
        span = [0]
        for b in bits[k + 1:]:
            span = span + [s ^ b for s in span]
        out.append(tuple(bits[k] ^ s for s in span))
    return tuple(out)


def _ag_send_sets(bits):
    held = [0]
    out = []
    for b in bits:
        out.append(tuple(held))
        held = held + [h ^ b for h in held]
    return tuple(out)


_RS_SENDS = tuple(_rs_send_sets(b) for b in _RS_BITS)
_AG_SENDS = tuple(_ag_send_sets(b) for b in _AG_BITS)
_RS_OFF = (0, 4, 6)
_AG_SEM_BASE = (0, 1, 3)


def _barrier(d):
    sem = pltpu.get_barrier_semaphore()
    for b in (1, 3, 4):
        pl.semaphore_signal(sem, inc=1, device_id=(jnp.bitwise_xor(d, b),),
                            device_id_type=pl.DeviceIdType.MESH)
    pl.semaphore_wait(sem, 3)


def _hd_allgather(out_ref, d, send_sems, recv_sems):
    for step in range(3):
        rdmas = []
        for sched, (c0, cw) in enumerate(_COLS):
            bit = _AG_BITS[sched][step]
            q = jnp.bitwise_xor(d, bit)
            for slot, j in enumerate(_AG_SENDS[sched][step]):
                c = jnp.bitwise_xor(d, j)
                sl = _AG_SEM_BASE[step] + slot
                rdma = pltpu.make_async_remote_copy(
                    src_ref=out_ref.at[pl.ds(c * M, M), pl.ds(c0, cw)],
                    dst_ref=out_ref.at[pl.ds(c * M, M), pl.ds(c0, cw)],
                    send_sem=send_sems.at[sched, sl],
                    recv_sem=recv_sems.at[sched, sl],
                    device_id=(q,),
                    device_id_type=pl.DeviceIdType.MESH,
                )
                rdma.start()
                rdmas.append(rdma)
        for r in rdmas:
            r.wait()


def _hd_reduce_scatter(acc_ref, snd_ref, rcv_ref, d, send_sems, recv_sems):
    for step in range(3):
        rdmas = []
        for sched, (c0, cw) in enumerate(_COLS):
            bit = _RS_BITS[sched][step]
            j_list = _RS_SENDS[sched][step]
            q = jnp.bitwise_xor(d, bit)
            off = _RS_OFF[step] * M
            for slot, j in enumerate(j_list):
                c = jnp.bitwise_xor(d, j)
                snd_ref[sched, pl.ds(off + slot * M, M), 0:cw] = acc_ref[
                    pl.ds(c * M, M), pl.ds(c0, cw)
                ].astype(jnp.bfloat16)
            nrows = len(j_list) * M
            rdma = pltpu.make_async_remote_copy(
                src_ref=snd_ref.at[sched, pl.ds(off, nrows), 0:cw],
                dst_ref=rcv_ref.at[sched, pl.ds(off, nrows), 0:cw],
                send_sem=send_sems.at[sched, step],
                recv_sem=recv_sems.at[sched, step],
                device_id=(q,),
                device_id_type=pl.DeviceIdType.MESH,
            )
            rdma.start()
            rdmas.append(rdma)
        for r in rdmas:
            r.wait()
        for sched, (c0, cw) in enumerate(_COLS):
            bit = _RS_BITS[sched][step]
            off = _RS_OFF[step] * M
            for slot, j in enumerate(_RS_SENDS[sched][step]):
                c = jnp.bitwise_xor(d, bit ^ j)
                acc_ref[pl.ds(c * M, M), pl.ds(c0, cw)] += rcv_ref[
                    sched, pl.ds(off + slot * M, M), 0:cw
                ].astype(jnp.float32)


def kernel(x, Win0, Wout0, Win1, Wout1, Win2, Wout2):
    m, n = x.shape
    B = N_DEV * m

    def body(x_ref, wi0, wo0, wi1, wo1, wi2, wo2, out_ref,
             xf_ref, acc_ref, wi_buf, wo_buf, snd_ref, rcv_ref,
             wi_sems, wo_sems, rs_ssem, rs_rsem, ag_ssem, ag_rsem):
        d = lax.axis_index(AXIS)
        if not _SKIP_COMM:
            _barrier(d)

        wins = [wi0, wi1, wi2]
        wouts = [wo0, wo1, wo2]
        copies = {}

        def issue(g):
            if g >= LAYERS * T:
                return
            L, t = divmod(g, T)
            s = g % SLOTS
            ci = pltpu.make_async_copy(
                wins[L].at[:, pl.ds(t * HT, HT)], wi_buf.at[s], wi_sems.at[s])
            co = pltpu.make_async_copy(
                wouts[L].at[pl.ds(t * HT, HT), :], wo_buf.at[s], wo_sems.at[s])
            ci.start()
            co.start()
            copies[g] = (ci, co)

        for g in range(SLOTS):
            issue(g)

        if _SKIP_COMM:
            for c in range(N_DEV):
                xf_ref[pl.ds(c * m, m), :] = x_ref[...].astype(jnp.bfloat16)
        else:
            xf_ref[pl.ds(d * m, m), :] = x_ref[...].astype(jnp.bfloat16)
            _hd_allgather(xf_ref, d, ag_ssem, ag_rsem)

        def load_cast(g):
            ci, co = copies.pop(g)
            ci.wait()
            co.wait()
            s = g % SLOTS
            return (wi_buf[s].astype(jnp.bfloat16),
                    wo_buf[s].astype(jnp.bfloat16))

        nxt = load_cast(0)
        for L in range(LAYERS):
            for t in range(T):
                g = L * T + t
                wbi, wbo = nxt
                h = jnp.dot(xf_ref[...], wbi,
                            preferred_element_type=jnp.float32)
                h = jnp.maximum(h, 0.0).astype(jnp.bfloat16)
                p = jnp.dot(h, wbo, preferred_element_type=jnp.float32)
                issue(g + SLOTS)
                if g + 1 < LAYERS * T:
                    nxt = load_cast(g + 1)
                if t == 0:
                    acc_ref[...] = p
                else:
                    acc_ref[...] += p

            if _SKIP_COMM:
                xf_ref[...] = acc_ref[...].astype(jnp.bfloat16)
            else:
                _hd_reduce_scatter(acc_ref, snd_ref, rcv_ref, d,
                                   rs_ssem, rs_rsem)
                xf_ref[pl.ds(d * M, M), :] = acc_ref[
                    pl.ds(d * M, M), :
                ].astype(jnp.bfloat16)
                _hd_allgather(xf_ref, d, ag_ssem, ag_rsem)

        out_ref[...] = xf_ref[...].astype(jnp.float32)

    any_spec = pl.BlockSpec(memory_space=pl.ANY)
    return pl.pallas_call(
        body,
        out_shape=jax.ShapeDtypeStruct((B, n), jnp.float32),
        in_specs=[pl.BlockSpec(memory_space=pltpu.VMEM)] + [any_spec] * 6,
        out_specs=pl.BlockSpec(memory_space=pltpu.VMEM),
        scratch_shapes=[
            pltpu.VMEM((B, n), jnp.bfloat16),
            pltpu.VMEM((B, n), jnp.float32),
            pltpu.VMEM((SLOTS, n, HT), jnp.float32),
            pltpu.VMEM((SLOTS, HT, n), jnp.float32),
            pltpu.VMEM((3, 7 * M, 768), jnp.bfloat16),
            pltpu.VMEM((3, 7 * M, 768), jnp.bfloat16),
            pltpu.SemaphoreType.DMA((SLOTS,)),
            pltpu.SemaphoreType.DMA((SLOTS,)),
            pltpu.SemaphoreType.DMA((3, 3)),
            pltpu.SemaphoreType.DMA((3, 3)),
            pltpu.SemaphoreType.DMA((3, 7)),
            pltpu.SemaphoreType.DMA((3, 7)),
        ],
        compiler_params=pltpu.CompilerParams(
            collective_id=None if _SKIP_COMM else 0,
            vmem_limit_bytes=60 * 1024 * 1024),
    )(x, Win0, Wout0, Win1, Wout1, Win2, Wout2)


# device time: 161671 ns/iter; 1.0146x vs baseline; 1.0117x over previous
import os

import jax
import jax.numpy as jnp
from jax import lax
from jax.experimental import pallas as pl
from jax.experimental.pallas import tpu as pltpu

_SKIP_COMM = os.environ.get("KERNEL_SKIP_COMM") == "1"

N_DEV = 8
AXIS = "i"
M = 64
LAYERS = 3
T = 8
HT = 512
SLOTS = 4

_COLS = ((0, 768), (768, 640), (1408, 640))

_RS_BITS = ((1, 3, 4), (3, 4, 1), (4, 1, 3))
_AG_BITS = ((4, 3, 1), (1, 4, 3), (3, 1, 4))


def _rs_send_sets(bits):
    out = []
    for k in range(3):
        span = [0]
        for b in bits[k + 1:]:
            span = span + [s ^ b for s in span]
        out.append(tuple(bits[k] ^ s for s in span))
    return tuple(out)


def _ag_send_sets(bits):
    held = [0]
    out = []
    for b in bits:
        out.append(tuple(held))
        held = held + [h ^ b for h in held]
    return tuple(out)


_RS_SENDS = tuple(_rs_send_sets(b) for b in _RS_BITS)
_AG_SENDS = tuple(_ag_send_sets(b) for b in _AG_BITS)
_RS_OFF = (0, 4, 6)
_AG_SEM_BASE = (0, 1, 3)


def _barrier(d):
    sem = pltpu.get_barrier_semaphore()
    for b in (1, 3, 4):
        pl.semaphore_signal(sem, inc=1, device_id=(jnp.bitwise_xor(d, b),),
                            device_id_type=pl.DeviceIdType.MESH)
    pl.semaphore_wait(sem, 3)


def _hd_allgather(out_ref, d, send_sems, recv_sems):
    for step in range(3):
        rdmas = []
        for sched, (c0, cw) in enumerate(_COLS):
            bit = _AG_BITS[sched][step]
            q = jnp.bitwise_xor(d, bit)
            for slot, j in enumerate(_AG_SENDS[sched][step]):
                c = jnp.bitwise_xor(d, j)
                sl = _AG_SEM_BASE[step] + slot
                rdma = pltpu.make_async_remote_copy(
                    src_ref=out_ref.at[pl.ds(c * M, M), pl.ds(c0, cw)],
                    dst_ref=out_ref.at[pl.ds(c * M, M), pl.ds(c0, cw)],
                    send_sem=send_sems.at[sched, sl],
                    recv_sem=recv_sems.at[sched, sl],
                    device_id=(q,),
                    device_id_type=pl.DeviceIdType.MESH,
                )
                rdma.start()
                rdmas.append(rdma)
        for r in rdmas:
            r.wait()


def _hd_reduce_scatter(acc_ref, snd_ref, rcv_ref, d, send_sems, recv_sems):
    for step in range(3):
        rdmas = []
        for sched, (c0, cw) in enumerate(_COLS):
            bit = _RS_BITS[sched][step]
            j_list = _RS_SENDS[sched][step]
            q = jnp.bitwise_xor(d, bit)
            off = _RS_OFF[step] * M
            for slot, j in enumerate(j_list):
                c = jnp.bitwise_xor(d, j)
                snd_ref[sched, pl.ds(off + slot * M, M), 0:cw] = acc_ref[
                    pl.ds(c * M, M), pl.ds(c0, cw)
                ].astype(jnp.bfloat16)
            nrows = len(j_list) * M
            rdma = pltpu.make_async_remote_copy(
                src_ref=snd_ref.at[sched, pl.ds(off, nrows), 0:cw],
                dst_ref=rcv_ref.at[sched, pl.ds(off, nrows), 0:cw],
                send_sem=send_sems.at[sched, step],
                recv_sem=recv_sems.at[sched, step],
                device_id=(q,),
                device_id_type=pl.DeviceIdType.MESH,
            )
            rdma.start()
            rdmas.append(rdma)
        for r in rdmas:
            r.wait()
        for sched, (c0, cw) in enumerate(_COLS):
            bit = _RS_BITS[sched][step]
            off = _RS_OFF[step] * M
            for slot, j in enumerate(_RS_SENDS[sched][step]):
                c = jnp.bitwise_xor(d, bit ^ j)
                acc_ref[pl.ds(c * M, M), pl.ds(c0, cw)] += rcv_ref[
                    sched, pl.ds(off + slot * M, M), 0:cw
                ].astype(jnp.float32)


def kernel(x, Win0, Wout0, Win1, Wout1, Win2, Wout2):
    m, n = x.shape
    B = N_DEV * m

    def body(x_ref, wi0, wo0, wi1, wo1, wi2, wo2, out_ref,
             xf_ref, acc_ref, wi_buf, wo_buf, snd_ref, rcv_ref,
             wi_sems, wo_sems, rs_ssem, rs_rsem, ag_ssem, ag_rsem):
        d = lax.axis_index(AXIS)
        if not _SKIP_COMM:
            _barrier(d)

        wins = [wi0, wi1, wi2]
        wouts = [wo0, wo1, wo2]
        copies = {}

        def issue(g):
            if g >= LAYERS * T:
                return
            L, t = divmod(g, T)
            s = g % SLOTS
            ci = pltpu.make_async_copy(
                wins[L].at[:, pl.ds(t * HT, HT)], wi_buf.at[s], wi_sems.at[s])
            co = pltpu.make_async_copy(
                wouts[L].at[pl.ds(t * HT, HT), :], wo_buf.at[s], wo_sems.at[s])
            ci.start()
            co.start()
            copies[g] = (ci, co)

        for g in range(SLOTS):
            issue(g)

        if _SKIP_COMM:
            for c in range(N_DEV):
                xf_ref[pl.ds(c * m, m), :] = x_ref[...].astype(jnp.bfloat16)
        else:
            xf_ref[pl.ds(d * m, m), :] = x_ref[...].astype(jnp.bfloat16)
            _hd_allgather(xf_ref, d, ag_ssem, ag_rsem)

        for L in range(LAYERS):
            for t in range(T):
                g = L * T + t
                ci, co = copies.pop(g)
                ci.wait()
                co.wait()
                s = g % SLOTS
                h = jnp.dot(xf_ref[...], wi_buf[s].astype(jnp.bfloat16),
                            preferred_element_type=jnp.float32)
                h = jnp.maximum(h, 0.0).astype(jnp.bfloat16)
                p = jnp.dot(h, wo_buf[s].astype(jnp.bfloat16),
                            preferred_element_type=jnp.float32)
                if t == 0:
                    acc_ref[...] = p
                else:
                    acc_ref[...] += p
                issue(g + SLOTS)

            if _SKIP_COMM:
                xf_ref[...] = acc_ref[...].astype(jnp.bfloat16)
            else:
                _hd_reduce_scatter(acc_ref, snd_ref, rcv_ref, d,
                                   rs_ssem, rs_rsem)
                xf_ref[pl.ds(d * M, M), :] = acc_ref[
                    pl.ds(d * M, M), :
                ].astype(jnp.bfloat16)
                _hd_allgather(xf_ref, d, ag_ssem, ag_rsem)

        out_ref[...] = xf_ref[...].astype(jnp.float32)

    any_spec = pl.BlockSpec(memory_space=pl.ANY)
    return pl.pallas_call(
        body,
        out_shape=jax.ShapeDtypeStruct((B, n), jnp.float32),
        in_specs=[pl.BlockSpec(memory_space=pltpu.VMEM)] + [any_spec] * 6,
        out_specs=pl.BlockSpec(memory_space=pltpu.VMEM),
        scratch_shapes=[
            pltpu.VMEM((B, n), jnp.bfloat16),
            pltpu.VMEM((B, n), jnp.float32),
            pltpu.VMEM((SLOTS, n, HT), jnp.float32),
            pltpu.VMEM((SLOTS, HT, n), jnp.float32),
            pltpu.VMEM((3, 7 * M, 768), jnp.bfloat16),
            pltpu.VMEM((3, 7 * M, 768), jnp.bfloat16),
            pltpu.SemaphoreType.DMA((SLOTS,)),
            pltpu.SemaphoreType.DMA((SLOTS,)),
            pltpu.SemaphoreType.DMA((3, 3)),
            pltpu.SemaphoreType.DMA((3, 3)),
            pltpu.SemaphoreType.DMA((3, 7)),
            pltpu.SemaphoreType.DMA((3, 7)),
        ],
        compiler_params=pltpu.CompilerParams(
            collective_id=None if _SKIP_COMM else 0,
            vmem_limit_bytes=60 * 1024 * 1024),
    )(x, Win0, Wout0, Win1, Wout1, Win2, Wout2)
